# baseline (speedup 1.0000x reference)
import functools

import jax
import jax.numpy as jnp
from jax import lax
from jax.experimental import pallas as pl
from jax.experimental.pallas import tpu as pltpu
from jax.experimental.pallas import tpu_sc as plsc

FIELDS = 8
B = 16384
V = 100000
D = 32
ND = 64
P = 128

_NC = 2
_NS = 16
_NW = _NC * _NS
_BPW = B // _NW
_R4 = V * D // 128

_VCH = 512
_NFULL = V // _VCH
_VTAIL = V - _NFULL * _VCH
_KMAX = -(-_NFULL // _NW)


def _make_sc_repack():
    mesh = plsc.VectorSubcoreMesh(core_axis_name="c", subcore_axis_name="s")

    @functools.partial(
        pl.kernel,
        mesh=mesh,
        out_type=[jax.ShapeDtypeStruct((_R4, 128), jnp.float32)
                  for _ in range(FIELDS)],
        scratch_types=[
            pltpu.VMEM((D, _VCH), jnp.float32),
            pltpu.VMEM((_VCH // 4, 128), jnp.float32),
            pltpu.SemaphoreType.DMA,
        ],
        compiler_params=pltpu.CompilerParams(needs_layout_passes=False),
    )
    def sc_repack(tt0, tt1, tt2, tt3, tt4, tt5, tt6, tt7,
                  tl0, tl1, tl2, tl3, tl4, tl5, tl6, tl7,
                  o0, o1, o2, o3, o4, o5, o6, o7, buf, tbuf, sem):
        wid = lax.axis_index("s") * _NC + lax.axis_index("c")
        tts = (tt0, tt1, tt2, tt3, tt4, tt5, tt6, tt7)
        tls = (tl0, tl1, tl2, tl3, tl4, tl5, tl6, tl7)
        outs = (o0, o1, o2, o3, o4, o5, o6, o7)
        iota = lax.iota(jnp.int32, 16)
        iota_hi = iota + 16
        for f in range(FIELDS):
            def chunk_body(k, _):
                c = wid + _NW * k

                @pl.when(c < _NFULL)
                def _():
                    s = c * _VCH
                    pltpu.sync_copy(tts[f].at[:, pl.ds(s, _VCH)], buf)

                    def vb_body(vb, _):
                        v0 = 8 * vb
                        for u in range(8):
                            v = v0 + u
                            cv = jnp.full((16,), v, jnp.int32)
                            a = plsc.load_gather(buf, [iota, cv])
                            b = plsc.load_gather(buf, [iota_hi, cv])
                            r = 2 * vb + (u // 4)
                            cb = 32 * (u % 4)
                            tbuf[r, pl.ds(cb, 16)] = a
                            tbuf[r, pl.ds(cb + 16, 16)] = b
                        return 0

                    lax.fori_loop(0, _VCH // 8, vb_body, 0)
                    pltpu.sync_copy(
                        tbuf, outs[f].at[pl.ds(c * (_VCH // 4), _VCH // 4), :])

                return 0

            lax.fori_loop(0, _KMAX, chunk_body, 0)

            @pl.when(wid == f + 8)
            def _():
                nt = _VTAIL * D // 128
                pltpu.sync_copy(tls[f], tbuf.at[pl.ds(0, nt), :])
                pltpu.sync_copy(tbuf.at[pl.ds(0, nt), :],
                                outs[f].at[pl.ds(_NFULL * (_VCH // 4), nt), :])

    return sc_repack


def _make_sc_gather():
    mesh = plsc.VectorSubcoreMesh(core_axis_name="c", subcore_axis_name="s")

    @functools.partial(
        pl.kernel,
        mesh=mesh,
        out_type=jax.ShapeDtypeStruct((FIELDS, B, D), jnp.float32),
        scratch_types=[
            pltpu.VMEM((_BPW,), jnp.int32),
            pltpu.VMEM((_BPW, D), jnp.float32),
            pltpu.SemaphoreType.DMA,
        ],
        compiler_params=pltpu.CompilerParams(use_tc_tiling_on_sc=False),
    )
    def sc_gather(i0, i1, i2, i3, i4, i5, i6, i7,
                  t0, t1, t2, t3, t4, t5, t6, t7,
                  out, idx_v, rows_v, sem):
        wid = lax.axis_index("s") * _NC + lax.axis_index("c")
        base = wid * _BPW
        idxs = (i0, i1, i2, i3, i4, i5, i6, i7)
        tabs = (t0, t1, t2, t3, t4, t5, t6, t7)
        for f in range(FIELDS):
            pltpu.sync_copy(idxs[f].at[pl.ds(base, _BPW)], idx_v)
            pltpu.async_copy(tabs[f].at[idx_v], rows_v, sem).wait()
            pltpu.sync_copy(rows_v, out.at[f].at[pl.ds(base, _BPW)])

    return sc_gather


_SC_REPACK = _make_sc_repack()
_SC_GATHER = _make_sc_gather()

_BS = 1024


def _tc_body(emb_ref, num_ref, g_ref, be_ref, wn_ref, bn_ref,
             wcat_ref, wnum_ref, bf_ref, out_ref):
    x = num_ref[...]
    mu = jnp.mean(x, axis=-1, keepdims=True)
    var = jnp.mean((x - mu) ** 2, axis=-1, keepdims=True)
    xn = (x - mu) * lax.rsqrt(var + 1e-5) * g_ref[...] + be_ref[...]
    nf = jnp.maximum(
        jnp.dot(xn, wn_ref[...], preferred_element_type=jnp.float32)
        + bn_ref[...], 0.0)
    emb = emb_ref[...]
    cat = jnp.concatenate([emb[f] for f in range(FIELDS)], axis=-1)
    acc = jnp.dot(cat, wcat_ref[...], preferred_element_type=jnp.float32)
    acc = acc + jnp.dot(nf, wnum_ref[...], preferred_element_type=jnp.float32)
    out_ref[...] = jnp.maximum(acc + bf_ref[...], 0.0)


def kernel(idx_0, idx_1, idx_2, idx_3, idx_4, idx_5, idx_6, idx_7,
           numeric_input,
           table_0, table_1, table_2, table_3, table_4, table_5, table_6,
           table_7, ln_gamma, ln_beta, W_num, b_num, W_final, b_final):
    tables = (table_0, table_1, table_2, table_3,
              table_4, table_5, table_6, table_7)
    tabs_t = [t.T for t in tables]
    tails = [t[_NFULL * _VCH:].reshape(_VTAIL * D // 128, 128)
             for t in tables]
    packed = _SC_REPACK(*tabs_t, *tails)
    emb = _SC_GATHER(idx_0, idx_1, idx_2, idx_3, idx_4, idx_5, idx_6, idx_7,
                     *[p.reshape(V, D) for p in packed])
    gam = ln_gamma.reshape(1, ND)
    bet = ln_beta.reshape(1, ND)
    bn = b_num.reshape(1, P)
    bf = b_final.reshape(1, P)
    wcat = W_final[:FIELDS * D]
    wnum = W_final[FIELDS * D:]
    out = pl.pallas_call(
        _tc_body,
        grid=(B // _BS,),
        in_specs=[
            pl.BlockSpec((FIELDS, _BS, D), lambda i: (0, i, 0)),
            pl.BlockSpec((_BS, ND), lambda i: (i, 0)),
            pl.BlockSpec((1, ND), lambda i: (0, 0)),
            pl.BlockSpec((1, ND), lambda i: (0, 0)),
            pl.BlockSpec((ND, P), lambda i: (0, 0)),
            pl.BlockSpec((1, P), lambda i: (0, 0)),
            pl.BlockSpec((FIELDS * D, P), lambda i: (0, 0)),
            pl.BlockSpec((P, P), lambda i: (0, 0)),
            pl.BlockSpec((1, P), lambda i: (0, 0)),
        ],
        out_specs=pl.BlockSpec((_BS, P), lambda i: (i, 0)),
        out_shape=jax.ShapeDtypeStruct((B, P), jnp.float32),
    )(emb, numeric_input, gam, bet, W_num, bn, wcat, wnum, bf)
    return out


# speedup vs baseline: 1.0607x; 1.0607x over previous
import functools

import jax
import jax.numpy as jnp
from jax import lax
from jax.experimental import pallas as pl
from jax.experimental.pallas import tpu as pltpu
from jax.experimental.pallas import tpu_sc as plsc

FIELDS = 8
B = 16384
V = 100000
D = 32
ND = 64
P = 128

_NC = 2
_NS = 16
_NW = _NC * _NS
_BPW = B // _NW
_R4 = V * D // 128

_VCH = 512
_NFULL = V // _VCH
_VTAIL = V - _NFULL * _VCH
_KMAX = -(-_NFULL // _NW)


def _make_sc_repack():
    mesh = plsc.VectorSubcoreMesh(core_axis_name="c", subcore_axis_name="s")

    @functools.partial(
        pl.kernel,
        mesh=mesh,
        out_type=[jax.ShapeDtypeStruct((_R4, 128), jnp.float32)
                  for _ in range(FIELDS)],
        scratch_types=[
            pltpu.VMEM((D, _VCH), jnp.float32),
            pltpu.VMEM((_VCH // 4, 128), jnp.float32),
            pltpu.SemaphoreType.DMA,
        ],
        compiler_params=pltpu.CompilerParams(needs_layout_passes=False),
    )
    def sc_repack(tt0, tt1, tt2, tt3, tt4, tt5, tt6, tt7,
                  tl0, tl1, tl2, tl3, tl4, tl5, tl6, tl7,
                  o0, o1, o2, o3, o4, o5, o6, o7, buf, tbuf, sem):
        wid = lax.axis_index("s") * _NC + lax.axis_index("c")
        tts = (tt0, tt1, tt2, tt3, tt4, tt5, tt6, tt7)
        tls = (tl0, tl1, tl2, tl3, tl4, tl5, tl6, tl7)
        outs = (o0, o1, o2, o3, o4, o5, o6, o7)
        iota = lax.iota(jnp.int32, 16)
        row_c = lax.shift_right_logical(iota, 2)
        col_c = lax.mul(jnp.bitwise_and(iota, 3), 32)
        for f in range(FIELDS):
            def chunk_body(k, _):
                c = wid + _NW * k

                @pl.when(c < _NFULL)
                def _():
                    s = c * _VCH
                    pltpu.sync_copy(tts[f].at[:, pl.ds(s, _VCH)], buf)

                    def d_body(d, _):
                        cols = col_c + d

                        def vb_body(vb, _):
                            rows = row_c + 4 * vb
                            vals = buf[d, pl.ds(16 * vb, 16)]
                            plsc.store_scatter(tbuf, [rows, cols], vals)
                            return 0

                        lax.fori_loop(0, _VCH // 16, vb_body, 0, unroll=8)
                        return 0

                    lax.fori_loop(0, D, d_body, 0)
                    pltpu.sync_copy(
                        tbuf, outs[f].at[pl.ds(c * (_VCH // 4), _VCH // 4), :])

                return 0

            lax.fori_loop(0, _KMAX, chunk_body, 0)

            @pl.when(wid == f + 8)
            def _():
                nt = _VTAIL * D // 128
                pltpu.sync_copy(tls[f], tbuf.at[pl.ds(0, nt), :])
                pltpu.sync_copy(tbuf.at[pl.ds(0, nt), :],
                                outs[f].at[pl.ds(_NFULL * (_VCH // 4), nt), :])

    return sc_repack


def _make_sc_gather():
    mesh = plsc.VectorSubcoreMesh(core_axis_name="c", subcore_axis_name="s")

    @functools.partial(
        pl.kernel,
        mesh=mesh,
        out_type=jax.ShapeDtypeStruct((FIELDS, B, D), jnp.float32),
        scratch_types=[
            pltpu.VMEM((_BPW,), jnp.int32),
            pltpu.VMEM((_BPW, D), jnp.float32),
            pltpu.SemaphoreType.DMA,
        ],
        compiler_params=pltpu.CompilerParams(use_tc_tiling_on_sc=False),
    )
    def sc_gather(i0, i1, i2, i3, i4, i5, i6, i7,
                  t0, t1, t2, t3, t4, t5, t6, t7,
                  out, idx_v, rows_v, sem):
        wid = lax.axis_index("s") * _NC + lax.axis_index("c")
        base = wid * _BPW
        idxs = (i0, i1, i2, i3, i4, i5, i6, i7)
        tabs = (t0, t1, t2, t3, t4, t5, t6, t7)
        for f in range(FIELDS):
            pltpu.sync_copy(idxs[f].at[pl.ds(base, _BPW)], idx_v)
            pltpu.async_copy(tabs[f].at[idx_v], rows_v, sem).wait()
            pltpu.sync_copy(rows_v, out.at[f].at[pl.ds(base, _BPW)])

    return sc_gather


_SC_REPACK = _make_sc_repack()
_SC_GATHER = _make_sc_gather()

_BS = 1024


def _tc_body(emb_ref, num_ref, g_ref, be_ref, wn_ref, bn_ref,
             wcat_ref, wnum_ref, bf_ref, out_ref):
    x = num_ref[...]
    mu = jnp.mean(x, axis=-1, keepdims=True)
    var = jnp.mean((x - mu) ** 2, axis=-1, keepdims=True)
    xn = (x - mu) * lax.rsqrt(var + 1e-5) * g_ref[...] + be_ref[...]
    nf = jnp.maximum(
        jnp.dot(xn, wn_ref[...], preferred_element_type=jnp.float32)
        + bn_ref[...], 0.0)
    emb = emb_ref[...]
    cat = jnp.concatenate([emb[f] for f in range(FIELDS)], axis=-1)
    acc = jnp.dot(cat, wcat_ref[...], preferred_element_type=jnp.float32)
    acc = acc + jnp.dot(nf, wnum_ref[...], preferred_element_type=jnp.float32)
    out_ref[...] = jnp.maximum(acc + bf_ref[...], 0.0)


def kernel(idx_0, idx_1, idx_2, idx_3, idx_4, idx_5, idx_6, idx_7,
           numeric_input,
           table_0, table_1, table_2, table_3, table_4, table_5, table_6,
           table_7, ln_gamma, ln_beta, W_num, b_num, W_final, b_final):
    tables = (table_0, table_1, table_2, table_3,
              table_4, table_5, table_6, table_7)
    tabs_t = [t.T for t in tables]
    tails = [t[_NFULL * _VCH:].reshape(_VTAIL * D // 128, 128)
             for t in tables]
    packed = _SC_REPACK(*tabs_t, *tails)
    emb = _SC_GATHER(idx_0, idx_1, idx_2, idx_3, idx_4, idx_5, idx_6, idx_7,
                     *[p.reshape(V, D) for p in packed])
    gam = ln_gamma.reshape(1, ND)
    bet = ln_beta.reshape(1, ND)
    bn = b_num.reshape(1, P)
    bf = b_final.reshape(1, P)
    wcat = W_final[:FIELDS * D]
    wnum = W_final[FIELDS * D:]
    out = pl.pallas_call(
        _tc_body,
        grid=(B // _BS,),
        in_specs=[
            pl.BlockSpec((FIELDS, _BS, D), lambda i: (0, i, 0)),
            pl.BlockSpec((_BS, ND), lambda i: (i, 0)),
            pl.BlockSpec((1, ND), lambda i: (0, 0)),
            pl.BlockSpec((1, ND), lambda i: (0, 0)),
            pl.BlockSpec((ND, P), lambda i: (0, 0)),
            pl.BlockSpec((1, P), lambda i: (0, 0)),
            pl.BlockSpec((FIELDS * D, P), lambda i: (0, 0)),
            pl.BlockSpec((P, P), lambda i: (0, 0)),
            pl.BlockSpec((1, P), lambda i: (0, 0)),
        ],
        out_specs=pl.BlockSpec((_BS, P), lambda i: (i, 0)),
        out_shape=jax.ShapeDtypeStruct((B, P), jnp.float32),
    )(emb, numeric_input, gam, bet, W_num, bn, wcat, wnum, bf)
    return out


# speedup vs baseline: 1.1385x; 1.0734x over previous
import functools

import jax
import jax.numpy as jnp
from jax import lax
from jax.experimental import pallas as pl
from jax.experimental.pallas import tpu as pltpu
from jax.experimental.pallas import tpu_sc as plsc

FIELDS = 8
B = 16384
V = 100000
D = 32
ND = 64
P = 128

_NC = 2
_NS = 16
_NW = _NC * _NS
_BPW = B // _NW

_VCH = 768
_NFULL = V // _VCH
_VTAIL = V - _NFULL * _VCH
_KMAX = -(-_NFULL // _NW)


def _make_sc_repack():
    mesh = plsc.VectorSubcoreMesh(core_axis_name="c", subcore_axis_name="s")

    @functools.partial(
        pl.kernel,
        mesh=mesh,
        out_type=[jax.ShapeDtypeStruct((V * D,), jnp.float32)
                  for _ in range(FIELDS)],
        scratch_types=[
            pltpu.VMEM((D, _VCH), jnp.float32),
            pltpu.VMEM((D, _VCH), jnp.float32),
            pltpu.VMEM((_VCH * D,), jnp.float32),
            pltpu.VMEM((_VCH * D,), jnp.float32),
            pltpu.SemaphoreType.DMA,
            pltpu.SemaphoreType.DMA,
            pltpu.SemaphoreType.DMA,
            pltpu.SemaphoreType.DMA,
        ],
        compiler_params=pltpu.CompilerParams(needs_layout_passes=False),
    )
    def sc_repack(tt0, tt1, tt2, tt3, tt4, tt5, tt6, tt7,
                  tl0, tl1, tl2, tl3, tl4, tl5, tl6, tl7,
                  o0, o1, o2, o3, o4, o5, o6, o7,
                  buf0, buf1, tb0, tb1, si0, si1, so0, so1):
        wid = lax.axis_index("s") * _NC + lax.axis_index("c")
        tts = (tt0, tt1, tt2, tt3, tt4, tt5, tt6, tt7)
        tls = (tl0, tl1, tl2, tl3, tl4, tl5, tl6, tl7)
        outs = (o0, o1, o2, o3, o4, o5, o6, o7)
        bufs = (buf0, buf1)
        tbs = (tb0, tb1)
        sis = (si0, si1)
        sos = (so0, so1)
        iota = lax.iota(jnp.int32, 16)
        w_c = lax.mul(iota, D)
        for f in range(FIELDS):
            @pl.when(wid < _NFULL)
            def _():
                pltpu.async_copy(tts[f].at[:, pl.ds(wid * _VCH, _VCH)],
                                 bufs[0], sis[0])

            for k in range(_KMAX):
                c = wid + _NW * k
                pb = k % 2

                @pl.when(c < _NFULL)
                def _(f=f, k=k, c=c, pb=pb):
                    pltpu.make_async_copy(
                        tts[f].at[:, pl.ds(c * _VCH, _VCH)],
                        bufs[pb], sis[pb]).wait()
                    if k + 1 < _KMAX:
                        @pl.when(c + _NW < _NFULL)
                        def _():
                            pltpu.async_copy(
                                tts[f].at[:, pl.ds((c + _NW) * _VCH, _VCH)],
                                bufs[1 - pb], sis[1 - pb])
                    if k >= 2:
                        pltpu.make_async_copy(
                            tbs[pb],
                            outs[f].at[pl.ds((c - 2 * _NW) * (_VCH * D),
                                             _VCH * D)],
                            sos[pb]).wait()

                    def d_body(d, _):
                        base = w_c + d

                        def vb_body(vb, _):
                            w = base + (16 * D) * vb
                            vals = bufs[pb][d, pl.ds(16 * vb, 16)]
                            plsc.store_scatter(tbs[pb], [w], vals)
                            return 0

                        lax.fori_loop(0, _VCH // 16, vb_body, 0, unroll=8)
                        return 0

                    lax.fori_loop(0, D, d_body, 0)
                    pltpu.async_copy(
                        tbs[pb],
                        outs[f].at[pl.ds(c * (_VCH * D), _VCH * D)],
                        sos[pb])

            for j in range(max(0, _KMAX - 2), _KMAX):
                c = wid + _NW * j

                @pl.when(c < _NFULL)
                def _(f=f, c=c, j=j):
                    pltpu.make_async_copy(
                        tbs[j % 2],
                        outs[f].at[pl.ds(c * (_VCH * D), _VCH * D)],
                        sos[j % 2]).wait()

            @pl.when(wid == f + 8)
            def _(f=f):
                nt = _VTAIL * D
                pltpu.sync_copy(tls[f], tb0.at[pl.ds(0, nt)])
                pltpu.sync_copy(tb0.at[pl.ds(0, nt)],
                                outs[f].at[pl.ds(_NFULL * _VCH * D, nt)])

    return sc_repack


def _make_sc_gather():
    mesh = plsc.VectorSubcoreMesh(core_axis_name="c", subcore_axis_name="s")

    @functools.partial(
        pl.kernel,
        mesh=mesh,
        out_type=jax.ShapeDtypeStruct((FIELDS, B, D), jnp.float32),
        scratch_types=[
            pltpu.VMEM((_BPW,), jnp.int32),
            pltpu.VMEM((_BPW, D), jnp.float32),
            pltpu.SemaphoreType.DMA,
        ],
        compiler_params=pltpu.CompilerParams(use_tc_tiling_on_sc=False),
    )
    def sc_gather(i0, i1, i2, i3, i4, i5, i6, i7,
                  t0, t1, t2, t3, t4, t5, t6, t7,
                  out, idx_v, rows_v, sem):
        wid = lax.axis_index("s") * _NC + lax.axis_index("c")
        base = wid * _BPW
        idxs = (i0, i1, i2, i3, i4, i5, i6, i7)
        tabs = (t0, t1, t2, t3, t4, t5, t6, t7)
        for f in range(FIELDS):
            pltpu.sync_copy(idxs[f].at[pl.ds(base, _BPW)], idx_v)
            pltpu.async_copy(tabs[f].at[idx_v], rows_v, sem).wait()
            pltpu.sync_copy(rows_v, out.at[f].at[pl.ds(base, _BPW)])

    return sc_gather


_SC_REPACK = _make_sc_repack()
_SC_GATHER = _make_sc_gather()

_BS = 1024


def _tc_body(emb_ref, num_ref, g_ref, be_ref, wn_ref, bn_ref,
             wcat_ref, wnum_ref, bf_ref, out_ref):
    x = num_ref[...]
    mu = jnp.mean(x, axis=-1, keepdims=True)
    var = jnp.mean((x - mu) ** 2, axis=-1, keepdims=True)
    xn = (x - mu) * lax.rsqrt(var + 1e-5) * g_ref[...] + be_ref[...]
    nf = jnp.maximum(
        jnp.dot(xn, wn_ref[...], preferred_element_type=jnp.float32)
        + bn_ref[...], 0.0)
    emb = emb_ref[...]
    cat = jnp.concatenate([emb[f] for f in range(FIELDS)], axis=-1)
    acc = jnp.dot(cat, wcat_ref[...], preferred_element_type=jnp.float32)
    acc = acc + jnp.dot(nf, wnum_ref[...], preferred_element_type=jnp.float32)
    out_ref[...] = jnp.maximum(acc + bf_ref[...], 0.0)


def kernel(idx_0, idx_1, idx_2, idx_3, idx_4, idx_5, idx_6, idx_7,
           numeric_input,
           table_0, table_1, table_2, table_3, table_4, table_5, table_6,
           table_7, ln_gamma, ln_beta, W_num, b_num, W_final, b_final):
    tables = (table_0, table_1, table_2, table_3,
              table_4, table_5, table_6, table_7)
    tabs_t = [t.T for t in tables]
    tails = [t[_NFULL * _VCH:].reshape(-1) for t in tables]
    packed = _SC_REPACK(*tabs_t, *tails)
    emb = _SC_GATHER(idx_0, idx_1, idx_2, idx_3, idx_4, idx_5, idx_6, idx_7,
                     *[p.reshape(V, D) for p in packed])
    gam = ln_gamma.reshape(1, ND)
    bet = ln_beta.reshape(1, ND)
    bn = b_num.reshape(1, P)
    bf = b_final.reshape(1, P)
    wcat = W_final[:FIELDS * D]
    wnum = W_final[FIELDS * D:]
    out = pl.pallas_call(
        _tc_body,
        grid=(B // _BS,),
        in_specs=[
            pl.BlockSpec((FIELDS, _BS, D), lambda i: (0, i, 0)),
            pl.BlockSpec((_BS, ND), lambda i: (i, 0)),
            pl.BlockSpec((1, ND), lambda i: (0, 0)),
            pl.BlockSpec((1, ND), lambda i: (0, 0)),
            pl.BlockSpec((ND, P), lambda i: (0, 0)),
            pl.BlockSpec((1, P), lambda i: (0, 0)),
            pl.BlockSpec((FIELDS * D, P), lambda i: (0, 0)),
            pl.BlockSpec((P, P), lambda i: (0, 0)),
            pl.BlockSpec((1, P), lambda i: (0, 0)),
        ],
        out_specs=pl.BlockSpec((_BS, P), lambda i: (i, 0)),
        out_shape=jax.ShapeDtypeStruct((B, P), jnp.float32),
    )(emb, numeric_input, gam, bet, W_num, bn, wcat, wnum, bf)
    return out


# speedup vs baseline: 1.9165x; 1.6834x over previous
import functools

import jax
import jax.numpy as jnp
from jax import lax
from jax.experimental import pallas as pl
from jax.experimental.pallas import tpu as pltpu
from jax.experimental.pallas import tpu_sc as plsc

FIELDS = 8
B = 16384
V = 100000
D = 32
ND = 64
P = 128

_NC = 2
_NS = 16
_NW = _NC * _NS
_BPW = B // _NW


def _make_sc_gather():
    mesh = plsc.VectorSubcoreMesh(core_axis_name="c", subcore_axis_name="s")

    @functools.partial(
        pl.kernel,
        mesh=mesh,
        out_type=jax.ShapeDtypeStruct((FIELDS, B, D), jnp.float32),
        scratch_types=[
            pltpu.VMEM((_BPW,), jnp.int32),
            pltpu.VMEM((_BPW, D), jnp.float32),
            pltpu.SemaphoreType.DMA,
        ],
        compiler_params=pltpu.CompilerParams(use_tc_tiling_on_sc=False),
    )
    def sc_gather(i0, i1, i2, i3, i4, i5, i6, i7,
                  t0, t1, t2, t3, t4, t5, t6, t7,
                  out, idx_v, rows_v, sem):
        wid = lax.axis_index("s") * _NC + lax.axis_index("c")
        base = wid * _BPW
        idxs = (i0, i1, i2, i3, i4, i5, i6, i7)
        tabs = (t0, t1, t2, t3, t4, t5, t6, t7)
        for f in range(FIELDS):
            pltpu.sync_copy(idxs[f].at[pl.ds(base, _BPW)], idx_v)
            pltpu.async_copy(tabs[f].at[idx_v], rows_v, sem).wait()
            pltpu.sync_copy(rows_v, out.at[f].at[pl.ds(base, _BPW)])

    return sc_gather


_SC_GATHER = _make_sc_gather()

_BS = 1024


def _tc_body(emb_ref, num_ref, g_ref, be_ref, wn_ref, bn_ref,
             wcat_ref, wnum_ref, bf_ref, out_ref):
    x = num_ref[...]
    mu = jnp.mean(x, axis=-1, keepdims=True)
    var = jnp.mean((x - mu) ** 2, axis=-1, keepdims=True)
    xn = (x - mu) * lax.rsqrt(var + 1e-5) * g_ref[...] + be_ref[...]
    nf = jnp.maximum(
        jnp.dot(xn, wn_ref[...], preferred_element_type=jnp.float32)
        + bn_ref[...], 0.0)
    emb = emb_ref[...]
    cat = jnp.concatenate([emb[f] for f in range(FIELDS)], axis=-1)
    acc = jnp.dot(cat, wcat_ref[...], preferred_element_type=jnp.float32)
    acc = acc + jnp.dot(nf, wnum_ref[...], preferred_element_type=jnp.float32)
    out_ref[...] = jnp.maximum(acc + bf_ref[...], 0.0)


def kernel(idx_0, idx_1, idx_2, idx_3, idx_4, idx_5, idx_6, idx_7,
           numeric_input,
           table_0, table_1, table_2, table_3, table_4, table_5, table_6,
           table_7, ln_gamma, ln_beta, W_num, b_num, W_final, b_final):
    emb = _SC_GATHER(idx_0, idx_1, idx_2, idx_3, idx_4, idx_5, idx_6, idx_7,
                     table_0, table_1, table_2, table_3, table_4, table_5,
                     table_6, table_7)
    gam = ln_gamma.reshape(1, ND)
    bet = ln_beta.reshape(1, ND)
    bn = b_num.reshape(1, P)
    bf = b_final.reshape(1, P)
    wcat = W_final[:FIELDS * D]
    wnum = W_final[FIELDS * D:]
    out = pl.pallas_call(
        _tc_body,
        grid=(B // _BS,),
        in_specs=[
            pl.BlockSpec((FIELDS, _BS, D), lambda i: (0, i, 0)),
            pl.BlockSpec((_BS, ND), lambda i: (i, 0)),
            pl.BlockSpec((1, ND), lambda i: (0, 0)),
            pl.BlockSpec((1, ND), lambda i: (0, 0)),
            pl.BlockSpec((ND, P), lambda i: (0, 0)),
            pl.BlockSpec((1, P), lambda i: (0, 0)),
            pl.BlockSpec((FIELDS * D, P), lambda i: (0, 0)),
            pl.BlockSpec((P, P), lambda i: (0, 0)),
            pl.BlockSpec((1, P), lambda i: (0, 0)),
        ],
        out_specs=pl.BlockSpec((_BS, P), lambda i: (i, 0)),
        out_shape=jax.ShapeDtypeStruct((B, P), jnp.float32),
    )(emb, numeric_input, gam, bet, W_num, bn, wcat, wnum, bf)
    return out


# speedup vs baseline: 1.9625x; 1.0240x over previous
import functools

import jax
import jax.numpy as jnp
from jax import lax
from jax.experimental import pallas as pl
from jax.experimental.pallas import tpu as pltpu
from jax.experimental.pallas import tpu_sc as plsc

FIELDS = 8
B = 16384
V = 100000
D = 32
ND = 64
P = 128

_NC = 2
_NS = 16
_NW = _NC * _NS
_BPW = B // _NW


def _make_sc_gather(nf):
    mesh = plsc.VectorSubcoreMesh(core_axis_name="c", subcore_axis_name="s")

    @functools.partial(
        pl.kernel,
        mesh=mesh,
        out_type=jax.ShapeDtypeStruct((nf, B, D), jnp.float32),
        scratch_types=[
            pltpu.VMEM((_BPW,), jnp.int32),
            pltpu.VMEM((_BPW, D), jnp.float32),
            pltpu.SemaphoreType.DMA,
        ],
        compiler_params=pltpu.CompilerParams(use_tc_tiling_on_sc=False),
    )
    def sc_gather(*refs):
        idxs = refs[:nf]
        tabs = refs[nf:2 * nf]
        out = refs[2 * nf]
        idx_v, rows_v, sem = refs[2 * nf + 1:]
        wid = lax.axis_index("s") * _NC + lax.axis_index("c")
        base = wid * _BPW
        for f in range(nf):
            pltpu.sync_copy(idxs[f].at[pl.ds(base, _BPW)], idx_v)
            pltpu.async_copy(tabs[f].at[idx_v], rows_v, sem).wait()
            pltpu.sync_copy(rows_v, out.at[f].at[pl.ds(base, _BPW)])

    return sc_gather


_SC_GATHER_H = _make_sc_gather(FIELDS // 2)

_BS = 1024


def _tc_body(emb_a_ref, emb_b_ref, num_ref, g_ref, be_ref, wn_ref, bn_ref,
             wcat_ref, wnum_ref, bf_ref, out_ref):
    x = num_ref[...]
    mu = jnp.mean(x, axis=-1, keepdims=True)
    var = jnp.mean((x - mu) ** 2, axis=-1, keepdims=True)
    xn = (x - mu) * lax.rsqrt(var + 1e-5) * g_ref[...] + be_ref[...]
    nf = jnp.maximum(
        jnp.dot(xn, wn_ref[...], preferred_element_type=jnp.float32)
        + bn_ref[...], 0.0)
    ea = emb_a_ref[...]
    eb = emb_b_ref[...]
    cat = jnp.concatenate(
        [ea[f] for f in range(FIELDS // 2)]
        + [eb[f] for f in range(FIELDS // 2)], axis=-1)
    acc = jnp.dot(cat, wcat_ref[...], preferred_element_type=jnp.float32)
    acc = acc + jnp.dot(nf, wnum_ref[...], preferred_element_type=jnp.float32)
    out_ref[...] = jnp.maximum(acc + bf_ref[...], 0.0)


def kernel(idx_0, idx_1, idx_2, idx_3, idx_4, idx_5, idx_6, idx_7,
           numeric_input,
           table_0, table_1, table_2, table_3, table_4, table_5, table_6,
           table_7, ln_gamma, ln_beta, W_num, b_num, W_final, b_final):
    emb_a = _SC_GATHER_H(idx_0, idx_1, idx_2, idx_3,
                         table_0, table_1, table_2, table_3)
    emb_b = _SC_GATHER_H(idx_4, idx_5, idx_6, idx_7,
                         table_4, table_5, table_6, table_7)
    gam = ln_gamma.reshape(1, ND)
    bet = ln_beta.reshape(1, ND)
    bn = b_num.reshape(1, P)
    bf = b_final.reshape(1, P)
    wcat = W_final[:FIELDS * D]
    wnum = W_final[FIELDS * D:]
    out = pl.pallas_call(
        _tc_body,
        grid=(B // _BS,),
        in_specs=[
            pl.BlockSpec((FIELDS // 2, _BS, D), lambda i: (0, i, 0)),
            pl.BlockSpec((FIELDS // 2, _BS, D), lambda i: (0, i, 0)),
            pl.BlockSpec((_BS, ND), lambda i: (i, 0)),
            pl.BlockSpec((1, ND), lambda i: (0, 0)),
            pl.BlockSpec((1, ND), lambda i: (0, 0)),
            pl.BlockSpec((ND, P), lambda i: (0, 0)),
            pl.BlockSpec((1, P), lambda i: (0, 0)),
            pl.BlockSpec((FIELDS * D, P), lambda i: (0, 0)),
            pl.BlockSpec((P, P), lambda i: (0, 0)),
            pl.BlockSpec((1, P), lambda i: (0, 0)),
        ],
        out_specs=pl.BlockSpec((_BS, P), lambda i: (i, 0)),
        out_shape=jax.ShapeDtypeStruct((B, P), jnp.float32),
    )(emb_a, emb_b, numeric_input, gam, bet, W_num, bn, wcat, wnum, bf)
    return out
